# Initial kernel scaffold; baseline (speedup 1.0000x reference)
import jax
import jax.numpy as jnp
from jax.experimental import pallas as pl


def kernel(x, wq_w, wv_w, wk, fanout_w, head_enabled, hcoo, n, layer, pas):
    raise NotImplementedError("write your pallas kernel here")


# speedup vs baseline: 1.2173x; 1.2173x over previous
import functools

import jax
import jax.numpy as jnp
from jax.experimental import pallas as pl
from jax.experimental.pallas import tpu as pltpu


TT = 256


def _attn_body(xbf_ref, xbt_ref, wqt_ref, wvt_ref, qs_ref, bqs_ref, bvh_ref,
               ap_ref, b_ref, vh_scr):
    h = pl.program_id(0)
    tb = pl.program_id(1)

    @pl.when(tb == 0)
    def _():
        vfull = jax.lax.dot_general(
            xbf_ref[:], wvt_ref[0],
            (((1,), (0,)), ((), ())),
            preferred_element_type=jnp.float32)
        vh_scr[:] = (vfull + bvh_ref[0]).astype(jnp.bfloat16)

    xt = xbf_ref[pl.ds(tb * TT, TT), :]
    q = jax.lax.dot_general(
        xt, wqt_ref[0], (((1,), (0,)), ((), ())),
        preferred_element_type=jnp.float32)
    qe = (q * qs_ref[0] + bqs_ref[0]).astype(jnp.bfloat16)

    logits = jax.lax.dot_general(
        qe, xbt_ref[:], (((1,), (0,)), ((), ())),
        preferred_element_type=jnp.float32)
    m = jnp.max(logits, axis=1, keepdims=True)
    e = jnp.exp(logits - m)
    s = jnp.sum(e, axis=1, keepdims=True)
    p = e * (1.0 / s)
    ap_ref[0] = p

    pv = jax.lax.dot_general(
        p.astype(jnp.bfloat16), vh_scr[:], (((1,), (0,)), ((), ())),
        preferred_element_type=jnp.float32)

    @pl.when(h == 0)
    def _():
        b_ref[pl.ds(tb * TT, TT), :] = pv

    @pl.when(h != 0)
    def _():
        b_ref[pl.ds(tb * TT, TT), :] += pv


def _fanout_body(b_ref, x_ref, wft_ref, bf_ref, o_ref):
    b = b_ref[:]
    g = b * jax.nn.sigmoid(1.702 * b)
    y = jax.lax.dot_general(
        g.astype(jnp.bfloat16), wft_ref[:], (((1,), (0,)), ((), ())),
        preferred_element_type=jnp.float32)
    o_ref[:] = x_ref[:] + y + bf_ref[0]


@functools.partial(jax.jit, static_argnums=(6, 7, 8, 9))
def _run(x, wq_w, wv_w, wk, fanout_w, head_enabled, B, T, D, H):
    x2 = x.reshape(T, D)
    xbf = x2.astype(jnp.bfloat16)
    xbt = xbf.T

    inv_sqrt_d = 1.0 / jnp.sqrt(jnp.float32(D))
    wqt = jnp.transpose(wq_w[:, :-1].reshape(H, D, D), (0, 2, 1)).astype(jnp.bfloat16)
    wvt = jnp.transpose((wv_w[:, :-1].reshape(H, D, D)
                         * head_enabled[:, None, None]), (0, 2, 1)).astype(jnp.bfloat16)
    qs = (wk * inv_sqrt_d).reshape(H, 1, D)
    bqs = (wq_w[:, -1].reshape(H, D) * qs.reshape(H, D)).reshape(H, 1, D)
    bvh = (wv_w[:, -1].reshape(H, D) * head_enabled[:, None]).reshape(H, 1, D)
    wft = fanout_w[:, :-1].T.astype(jnp.bfloat16)
    bf = fanout_w[:, -1].reshape(1, D)

    ap_htt, bsum = pl.pallas_call(
        _attn_body,
        grid=(H, T // TT),
        in_specs=[
            pl.BlockSpec((T, D), lambda h, tb: (0, 0)),
            pl.BlockSpec((D, T), lambda h, tb: (0, 0)),
            pl.BlockSpec((1, D, D), lambda h, tb: (h, 0, 0)),
            pl.BlockSpec((1, D, D), lambda h, tb: (h, 0, 0)),
            pl.BlockSpec((1, 1, D), lambda h, tb: (h, 0, 0)),
            pl.BlockSpec((1, 1, D), lambda h, tb: (h, 0, 0)),
            pl.BlockSpec((1, 1, D), lambda h, tb: (h, 0, 0)),
        ],
        out_specs=[
            pl.BlockSpec((1, TT, T), lambda h, tb: (h, tb, 0)),
            pl.BlockSpec((T, D), lambda h, tb: (0, 0)),
        ],
        out_shape=[
            jax.ShapeDtypeStruct((H, T, T), jnp.float32),
            jax.ShapeDtypeStruct((T, D), jnp.float32),
        ],
        scratch_shapes=[pltpu.VMEM((T, D), jnp.bfloat16)],
    )(xbf, xbt, wqt, wvt, qs, bqs, bvh)

    out1 = pl.pallas_call(
        _fanout_body,
        grid=(T // TT,),
        in_specs=[
            pl.BlockSpec((TT, D), lambda tb: (tb, 0)),
            pl.BlockSpec((TT, D), lambda tb: (tb, 0)),
            pl.BlockSpec((D, D), lambda tb: (0, 0)),
            pl.BlockSpec((1, D), lambda tb: (0, 0)),
        ],
        out_specs=pl.BlockSpec((TT, D), lambda tb: (tb, 0)),
        out_shape=jax.ShapeDtypeStruct((T, D), jnp.float32),
    )(bsum, x2, wft, bf)

    ap = jnp.transpose(ap_htt, (1, 2, 0))
    return out1.reshape(B, T, D), ap


def kernel(x, wq_w, wv_w, wk, fanout_w, head_enabled, hcoo, n, layer, pas):
    B, T, D = x.shape
    H = wk.shape[0]
    return _run(x, wq_w, wv_w, wk, fanout_w, head_enabled, B, T, D, H)
